# baseline (speedup 1.0000x reference)
import functools

import jax
import jax.numpy as jnp
from jax import lax
from jax.experimental import pallas as pl
from jax.experimental.pallas import tpu as pltpu
from jax.experimental.pallas import tpu_sc as plsc

_B = 16384
_J = 10
_K = 16
_NC = 2
_NS = 16
_NW = _NC * _NS
_BPW = _B // _NW
_CHJ = 5


def _sc_body(x_hbm, out_hbm, xv0, xv1, bestv, bjvv, outv, sem0, sem1):
    c = lax.axis_index("c")
    s = lax.axis_index("s")
    wid = s * _NC + c
    base = wid * _BPW

    bufs = (xv0, xv1)

    h0 = pltpu.async_copy(
        x_hbm.at[pl.ds(0, _CHJ * _K), pl.ds(base, _BPW)], xv0, sem0
    )
    h1 = pltpu.async_copy(
        x_hbm.at[pl.ds(_CHJ * _K, _CHJ * _K), pl.ds(base, _BPW)], xv1, sem1
    )

    def _compute(half):
        buf = bufs[half]
        first = half == 0
        last = half == 1

        def _block(g, carry):
            col = g * 16
            if first:
                best = jnp.full((16,), -1.0, jnp.float32)
                bjv = jnp.zeros((16,), jnp.int32)
            else:
                best = bestv[pl.ds(col, 16)]
                bjv = bjvv[pl.ds(col, 16)]
            for jj in range(_CHJ):
                j = half * _CHJ + jj
                acc = None
                for k in range(_K):
                    v = buf[jj * _K + k, pl.ds(col, 16)]
                    sq = v * v
                    acc = sq if acc is None else acc + sq
                p = acc > best
                best = jnp.where(p, acc, best)
                bjv = jnp.where(p, jnp.int32(j), bjv)
            if not last:
                bestv[pl.ds(col, 16)] = best
                bjvv[pl.ds(col, 16)] = bjv
            else:
                q = g // 8
                bi = (g % 8) * 16
                for j in range(16):
                    if j < _J:
                        vec = jnp.where(
                            bjv == j, jnp.float32(1.0), jnp.float32(0.0)
                        )
                    else:
                        vec = jnp.zeros((16,), jnp.float32)
                    outv[j // 8, q, j % 8, pl.ds(bi, 16)] = vec
            return carry

        lax.fori_loop(0, _BPW // 16, _block, 0)

    h0.wait()
    _compute(0)
    h1.wait()
    _compute(1)

    pltpu.sync_copy(outv, out_hbm.at[:, pl.ds(base // 128, _BPW // 128), :, :])


_decoder_sc = functools.partial(
    pl.kernel,
    mesh=plsc.VectorSubcoreMesh(core_axis_name="c", subcore_axis_name="s"),
    out_type=jax.ShapeDtypeStruct((2, _B // 128, 8, 128), jnp.float32),
    scratch_types=[
        pltpu.VMEM((_CHJ * _K, _BPW), jnp.float32),
        pltpu.VMEM((_CHJ * _K, _BPW), jnp.float32),
        pltpu.VMEM((_BPW,), jnp.float32),
        pltpu.VMEM((_BPW,), jnp.int32),
        pltpu.VMEM((2, _BPW // 128, 8, 128), jnp.float32),
        pltpu.SemaphoreType.DMA,
        pltpu.SemaphoreType.DMA,
    ],
    compiler_params=pltpu.CompilerParams(
        needs_layout_passes=False,
        use_tc_tiling_on_sc=False,
    ),
)(_sc_body)


def kernel(x, data):
    del data
    xt = jnp.transpose(x, (1, 2, 3, 0)).reshape(_J * _K, _B)
    o = _decoder_sc(xt)
    return o.transpose(1, 3, 0, 2).reshape(_B, 16)[:, :_J]
